# baseline (speedup 1.0000x reference)
import jax
import jax.numpy as jnp
from jax import lax
from jax.experimental import pallas as pl
from jax.experimental.pallas import tpu as pltpu
from jax.experimental.pallas import tpu_sc as plsc

EMB = 128
MARGIN = 0.1
INF = 5.0
B = 4096
L = 16
NW = 32
BPT = B // NW
NBLK = BPT // L
NC1 = 5
NC2 = 6
NR = 4


def _sqrt16(s):
    i = plsc.bitcast(s, jnp.int32)
    y = plsc.bitcast(jnp.int32(0x5F3759DF) - lax.shift_right_arithmetic(i, 1),
                     jnp.float32)
    for _ in range(3):
        y = y * (1.5 - ((0.5 * s) * y) * y)
    return s * y


def _relu(x):
    return jnp.maximum(x, 0.0)


def _sc_body(xs_hbm, ts_hbm, rel_hbm, nf1_h, nf2_h, nf3_h, nf4_h, top_h,
             nn_h, rad_h, out_h,
             nf1_v, nf2_v, nf3_v, nf4_v, nn_v, top_v, rad_v,
             ic1a, ic1b, ic2a, ic2b, ira, irb,
             c1a, c1b, c2a, c2b, ra, rb,
             tc_v, ta_v, tot_v, sem, tsem):
    wid = lax.axis_index("s") * 2 + lax.axis_index("c")
    base = wid * BPT
    iota = lax.iota(jnp.int32, L)

    pltpu.sync_copy(nf1_h.at[pl.ds(base * 3, BPT * 3)], nf1_v)
    pltpu.sync_copy(nf2_h.at[pl.ds(base * 3, BPT * 3)], nf2_v)
    pltpu.sync_copy(nf3_h.at[pl.ds(base * 3, BPT * 3)], nf3_v)
    pltpu.sync_copy(nf4_h.at[pl.ds(base * 3, BPT * 3)], nf4_v)
    pltpu.sync_copy(nn_h.at[pl.ds(base * 3, BPT * 3)], nn_v)
    pltpu.sync_copy(top_h.at[pl.ds(base, BPT)], top_v)
    pltpu.sync_copy(rad_h.at[pl.ds(base, BPT)], rad_v)

    col_specs = ((nf1_v, 0), (nf1_v, 2), (nf2_v, 0), (nf2_v, 1), (nf2_v, 2),
                 (nf3_v, 0), (nf3_v, 2), (nf4_v, 1), (nf4_v, 2),
                 (nn_v, 0), (nn_v, 2))
    for s, (ref, c) in enumerate(col_specs):
        for b8 in range(NBLK):
            tc_v[s, pl.ds(b8 * L, L)] = plsc.load_gather(
                ref, [(iota + b8 * L) * 3 + c])

    ic1 = (ic1a, ic1b)
    ic2 = (ic2a, ic2b)
    ir = (ira, irb)
    c1 = (c1a, c1b)
    c2 = (c2a, c2b)
    r = (ra, rb)

    def build_idx(blk, which):
        rows = iota + blk * L

        def col(ref, c):
            return plsc.load_gather(ref, [rows * 3 + c])

        ic1[which][pl.ds(0 * L, L)] = col(nf1_v, 0)
        ic1[which][pl.ds(1 * L, L)] = col(nf1_v, 2)
        ic1[which][pl.ds(2 * L, L)] = col(nf2_v, 0)
        ic1[which][pl.ds(3 * L, L)] = col(nf2_v, 1)
        ic1[which][pl.ds(4 * L, L)] = col(nf2_v, 2)
        ic2[which][pl.ds(0 * L, L)] = col(nf3_v, 0)
        ic2[which][pl.ds(1 * L, L)] = col(nf3_v, 2)
        ic2[which][pl.ds(2 * L, L)] = col(nf4_v, 1)
        ic2[which][pl.ds(3 * L, L)] = col(nf4_v, 2)
        ic2[which][pl.ds(4 * L, L)] = col(nn_v, 0)
        ic2[which][pl.ds(5 * L, L)] = col(nn_v, 2)
        ir[which][pl.ds(0 * L, L)] = col(nf1_v, 1)
        ir[which][pl.ds(1 * L, L)] = col(nf3_v, 1)
        ir[which][pl.ds(2 * L, L)] = col(nf4_v, 0)
        ir[which][pl.ds(3 * L, L)] = col(nn_v, 1)

    def issue(which):
        pltpu.async_copy(xs_hbm.at[ic1[which], pl.ds(0, EMB)], c1[which], sem)
        pltpu.async_copy(xs_hbm.at[ic2[which], pl.ds(0, EMB)], c2[which], sem)
        pltpu.async_copy(rel_hbm.at[ir[which]], r[which], sem)

    def drain(which):
        pltpu.make_async_copy(xs_hbm.at[ic1[which], pl.ds(0, EMB)], c1[which],
                              sem).wait()
        pltpu.make_async_copy(xs_hbm.at[ic2[which], pl.ds(0, EMB)], c2[which],
                              sem).wait()
        pltpu.make_async_copy(rel_hbm.at[ir[which]], r[which], sem).wait()

    build_idx(0, 0)
    issue(0)
    tdmas = [pltpu.async_copy(ts_hbm.at[tc_v.at[s]], ta_v.at[s], tsem)
             for s in range(11)]
    tdmas.append(pltpu.async_copy(ts_hbm.at[top_v], ta_v.at[11], tsem))
    tdmas.append(pltpu.async_copy(ts_hbm.at[rad_v], ta_v.at[12], tsem))
    for d in tdmas:
        d.wait()

    tot_v[...] = jnp.zeros((L,), jnp.float32)

    @pl.loop(0, NBLK, step=2)
    def _blockpair(g0):
        for bsel in range(2):
            blk = g0 + bsel
            nxt = 1 - bsel
            drain(bsel)
            if bsel == 0:
                build_idx(blk + 1, nxt)
                issue(nxt)
            else:
                @pl.when(g0 + 2 < NBLK)
                def _():
                    build_idx(blk + 1, nxt)
                    issue(nxt)

            zero = jnp.zeros((L,), jnp.float32)
            c1b_ = c1[bsel]
            c2b_ = c2[bsel]
            rb_ = r[bsel]
            row = [iota + s * L for s in range(6)]

            def eb1(e, accs):
                (aA, aB, aC, aD, aE, a1, aCD, aCE, aDE) = accs
                ce = (jnp.full((L,), e, jnp.int32) + iota) & (EMB - 1)
                vA = plsc.load_gather(c1b_, [row[0], ce])
                vB = plsc.load_gather(c1b_, [row[1], ce])
                vC = plsc.load_gather(c1b_, [row[2], ce])
                vD = plsc.load_gather(c1b_, [row[3], ce])
                vE = plsc.load_gather(c1b_, [row[4], ce])
                w1 = plsc.load_gather(rb_, [row[0], ce])
                aA = aA + vA * vA
                aB = aB + vB * vB
                aC = aC + vC * vC
                aD = aD + vD * vD
                aE = aE + vE * vE
                t = vA + w1 - vB
                a1 = a1 + t * t
                t = vD - vC
                aCD = aCD + t * t
                t = vE - vC
                aCE = aCE + t * t
                t = vE - vD
                aDE = aDE + t * t
                return (aA, aB, aC, aD, aE, a1, aCD, aCE, aDE)

            (aA, aB, aC, aD, aE, a1, aCD, aCE, aDE) = plsc.parallel_loop(
                0, EMB, unroll=4, carry=(zero,) * 9)(eb1)

            def eb2(e, accs):
                (aF, aG, aH, aI, aK, aL, a3, a4, a5) = accs
                ce = (jnp.full((L,), e, jnp.int32) + iota) & (EMB - 1)
                vF = plsc.load_gather(c2b_, [row[0], ce])
                vG = plsc.load_gather(c2b_, [row[1], ce])
                vH = plsc.load_gather(c2b_, [row[2], ce])
                vI = plsc.load_gather(c2b_, [row[3], ce])
                vK = plsc.load_gather(c2b_, [row[4], ce])
                vL = plsc.load_gather(c2b_, [row[5], ce])
                w3 = plsc.load_gather(rb_, [row[1], ce])
                w4 = plsc.load_gather(rb_, [row[2], ce])
                w5 = plsc.load_gather(rb_, [row[3], ce])
                aF = aF + vF * vF
                aG = aG + vG * vG
                aH = aH + vH * vH
                aI = aI + vI * vI
                aK = aK + vK * vK
                aL = aL + vL * vL
                t = vF + w3 - vG
                a3 = a3 + t * t
                t = vH - w4 - vI
                a4 = a4 + t * t
                t = vK + w5 - vL
                a5 = a5 + t * t
                return (aF, aG, aH, aI, aK, aL, a3, a4, a5)

            (aF, aG, aH, aI, aK, aL, a3, a4, a5) = plsc.parallel_loop(
                0, EMB, unroll=4, carry=(zero,) * 9)(eb2)

            bs = pl.ds(blk * L, L)
            rA = _relu(ta_v[0, bs]); rB = _relu(ta_v[1, bs])
            rC = _relu(ta_v[2, bs]); rD = _relu(ta_v[3, bs])
            rE = _relu(ta_v[4, bs]); rF = _relu(ta_v[5, bs])
            rG = _relu(ta_v[6, bs]); rH = _relu(ta_v[7, bs])
            rI = _relu(ta_v[8, bs]); rK = _relu(ta_v[9, bs])
            rL = _relu(ta_v[10, bs]); rJ = _relu(ta_v[11, bs])
            tP = ta_v[12, bs]

            def reg(a):
                return jnp.abs(_sqrt16(a) - 1.0)

            loss = (
                _relu(_sqrt16(a1) + rA - rB - MARGIN) + reg(aA) + reg(aB)
                + _relu(_sqrt16(aCD) - (rC + rD) - MARGIN)
                + _relu(_sqrt16(aCE) - rC - MARGIN)
                + _relu(_sqrt16(aDE) - rD - MARGIN)
                + _relu(jnp.minimum(rC, rD) - rE - MARGIN)
                + reg(aC) + reg(aD) + reg(aE)
                + _relu(_sqrt16(a3) + rF - rG - MARGIN) + reg(aF) + reg(aG)
                + _relu(_sqrt16(a4) - (rH + rI) - MARGIN) + reg(aH) + reg(aI)
                + jnp.abs(rJ - INF)
                + (MARGIN - (_sqrt16(a5) - rK - rL)) + reg(aK) + reg(aL)
                - jnp.minimum(tP, 0.0)
            )
            tot_v[...] = tot_v[...] + loss

    pltpu.sync_copy(tot_v, out_h.at[pl.ds(wid * L, L)])


def kernel(cls_emb, rel_emb, nf1, nf2, nf3, nf4, dis, top, nf3_neg,
           nf_inclusion, nf_chain, radius, dataset):
    ts = cls_emb[:, EMB]
    mesh = plsc.VectorSubcoreMesh(core_axis_name="c", subcore_axis_name="s")
    cp = pltpu.CompilerParams(needs_layout_passes=False,
                              use_tc_tiling_on_sc=True)
    sc = pl.kernel(
        _sc_body,
        out_type=jax.ShapeDtypeStruct((NW * L,), jnp.float32),
        mesh=mesh,
        compiler_params=cp,
        scratch_types=[
            pltpu.VMEM((BPT * 3,), jnp.int32),
            pltpu.VMEM((BPT * 3,), jnp.int32),
            pltpu.VMEM((BPT * 3,), jnp.int32),
            pltpu.VMEM((BPT * 3,), jnp.int32),
            pltpu.VMEM((BPT * 3,), jnp.int32),
            pltpu.VMEM((BPT,), jnp.int32),
            pltpu.VMEM((BPT,), jnp.int32),
            pltpu.VMEM((NC1 * L,), jnp.int32),
            pltpu.VMEM((NC1 * L,), jnp.int32),
            pltpu.VMEM((NC2 * L,), jnp.int32),
            pltpu.VMEM((NC2 * L,), jnp.int32),
            pltpu.VMEM((NR * L,), jnp.int32),
            pltpu.VMEM((NR * L,), jnp.int32),
            pltpu.VMEM((NC1 * L, EMB), jnp.float32),
            pltpu.VMEM((NC1 * L, EMB), jnp.float32),
            pltpu.VMEM((NC2 * L, EMB), jnp.float32),
            pltpu.VMEM((NC2 * L, EMB), jnp.float32),
            pltpu.VMEM((NR * L, EMB), jnp.float32),
            pltpu.VMEM((NR * L, EMB), jnp.float32),
            pltpu.VMEM((11, BPT), jnp.int32),
            pltpu.VMEM((13, BPT), jnp.float32),
            pltpu.VMEM((L,), jnp.float32),
            pltpu.SemaphoreType.DMA,
            pltpu.SemaphoreType.DMA,
        ],
    )
    part = sc(cls_emb, ts, rel_emb,
              nf1.reshape(-1).astype(jnp.int32),
              nf2.reshape(-1).astype(jnp.int32),
              nf3.reshape(-1).astype(jnp.int32),
              nf4.reshape(-1).astype(jnp.int32),
              top.astype(jnp.int32), nf3_neg.reshape(-1).astype(jnp.int32),
              radius.astype(jnp.int32))
    return (jnp.sum(part) / jnp.float32(B)) ** 2


# speedup vs baseline: 1.0295x; 1.0295x over previous
import jax
import jax.numpy as jnp
from jax import lax
from jax.experimental import pallas as pl
from jax.experimental.pallas import tpu as pltpu
from jax.experimental.pallas import tpu_sc as plsc

EMB = 128
MARGIN = 0.1
INF = 5.0
B = 4096
L = 16
NW = 32
BPT = B // NW
NBLK = BPT // L
NC1 = 5
NC2 = 6
NR = 4


def _sqrt16(s):
    i = plsc.bitcast(s, jnp.int32)
    y = plsc.bitcast(jnp.int32(0x5F3759DF) - lax.shift_right_arithmetic(i, 1),
                     jnp.float32)
    for _ in range(3):
        y = y * (1.5 - ((0.5 * s) * y) * y)
    return s * y


def _relu(x):
    return jnp.maximum(x, 0.0)


def _sc_body(xs_hbm, ts_hbm, rel_hbm, nf1_h, nf2_h, nf3_h, nf4_h, top_h,
             nn_h, rad_h, out_h,
             nf1_v, nf2_v, nf3_v, nf4_v, nn_v, top_v, rad_v,
             ic1_v, ic2_v, ir_v, c1_v, c2_v, r_v,
             tc_v, ta_v, tot_v, sem, tsem):
    wid = lax.axis_index("s") * 2 + lax.axis_index("c")
    base = wid * BPT
    iota = lax.iota(jnp.int32, L)

    pltpu.sync_copy(nf1_h.at[pl.ds(base, BPT), :], nf1_v)
    pltpu.sync_copy(nf2_h.at[pl.ds(base, BPT), :], nf2_v)
    pltpu.sync_copy(nf3_h.at[pl.ds(base, BPT), :], nf3_v)
    pltpu.sync_copy(nf4_h.at[pl.ds(base, BPT), :], nf4_v)
    pltpu.sync_copy(nn_h.at[pl.ds(base, BPT), :], nn_v)
    pltpu.sync_copy(top_h.at[pl.ds(base, BPT)], top_v)
    pltpu.sync_copy(rad_h.at[pl.ds(base, BPT)], rad_v)

    col_specs = ((nf1_v, 0), (nf1_v, 2), (nf2_v, 0), (nf2_v, 1), (nf2_v, 2),
                 (nf3_v, 0), (nf3_v, 2), (nf4_v, 1), (nf4_v, 2),
                 (nn_v, 0), (nn_v, 2))
    for s, (ref, c) in enumerate(col_specs):
        cc = jnp.full((L,), c, jnp.int32)
        for b8 in range(NBLK):
            tc_v[s, pl.ds(b8 * L, L)] = plsc.load_gather(
                ref, [iota + b8 * L, cc])

    def build_idx(blk, which):
        rows = iota + blk * L

        def col(ref, c):
            return plsc.load_gather(ref, [rows, jnp.full((L,), c, jnp.int32)])

        ic1_v[which, pl.ds(0 * L, L)] = col(nf1_v, 0)
        ic1_v[which, pl.ds(1 * L, L)] = col(nf1_v, 2)
        ic1_v[which, pl.ds(2 * L, L)] = col(nf2_v, 0)
        ic1_v[which, pl.ds(3 * L, L)] = col(nf2_v, 1)
        ic1_v[which, pl.ds(4 * L, L)] = col(nf2_v, 2)
        ic2_v[which, pl.ds(0 * L, L)] = col(nf3_v, 0)
        ic2_v[which, pl.ds(1 * L, L)] = col(nf3_v, 2)
        ic2_v[which, pl.ds(2 * L, L)] = col(nf4_v, 1)
        ic2_v[which, pl.ds(3 * L, L)] = col(nf4_v, 2)
        ic2_v[which, pl.ds(4 * L, L)] = col(nn_v, 0)
        ic2_v[which, pl.ds(5 * L, L)] = col(nn_v, 2)
        ir_v[which, pl.ds(0 * L, L)] = col(nf1_v, 1)
        ir_v[which, pl.ds(1 * L, L)] = col(nf3_v, 1)
        ir_v[which, pl.ds(2 * L, L)] = col(nf4_v, 0)
        ir_v[which, pl.ds(3 * L, L)] = col(nn_v, 1)

    def issue(which):
        pltpu.async_copy(xs_hbm.at[ic1_v.at[which]], c1_v.at[which], sem)
        pltpu.async_copy(xs_hbm.at[ic2_v.at[which]], c2_v.at[which], sem)
        pltpu.async_copy(rel_hbm.at[ir_v.at[which]], r_v.at[which], sem)

    def drain(which):
        pltpu.make_async_copy(xs_hbm.at[ic1_v.at[which]], c1_v.at[which],
                              sem).wait()
        pltpu.make_async_copy(xs_hbm.at[ic2_v.at[which]], c2_v.at[which],
                              sem).wait()
        pltpu.make_async_copy(rel_hbm.at[ir_v.at[which]], r_v.at[which],
                              sem).wait()

    build_idx(0, 0)
    issue(0)
    tdmas = [pltpu.async_copy(ts_hbm.at[tc_v.at[s]], ta_v.at[s], tsem)
             for s in range(11)]
    tdmas.append(pltpu.async_copy(ts_hbm.at[top_v], ta_v.at[11], tsem))
    tdmas.append(pltpu.async_copy(ts_hbm.at[rad_v], ta_v.at[12], tsem))

    def drain_t():
        for s in range(11):
            pltpu.make_async_copy(ts_hbm.at[tc_v.at[s]], ta_v.at[s],
                                  tsem).wait()
        pltpu.make_async_copy(ts_hbm.at[top_v], ta_v.at[11], tsem).wait()
        pltpu.make_async_copy(ts_hbm.at[rad_v], ta_v.at[12], tsem).wait()

    tot_v[...] = jnp.zeros((L,), jnp.float32)

    @pl.loop(0, NBLK, step=2)
    def _blockpair(g0):
        for bsel in range(2):
            blk = g0 + bsel
            nxt = 1 - bsel
            drain(bsel)
            if bsel == 0:
                build_idx(blk + 1, nxt)
                issue(nxt)
            else:
                @pl.when(g0 + 2 < NBLK)
                def _():
                    build_idx(blk + 1, nxt)
                    issue(nxt)

            zero = jnp.zeros((L,), jnp.float32)
            c1b = c1_v.at[bsel]
            c2b = c2_v.at[bsel]
            rb = r_v.at[bsel]
            row = [iota + s * L for s in range(6)]

            def eb1(e, accs):
                (aA, aB, aC, aD, aE, a1, aCD, aCE, aDE) = accs
                ce = (jnp.full((L,), e, jnp.int32) + iota) & (EMB - 1)
                vA = plsc.load_gather(c1b, [row[0], ce])
                vB = plsc.load_gather(c1b, [row[1], ce])
                vC = plsc.load_gather(c1b, [row[2], ce])
                vD = plsc.load_gather(c1b, [row[3], ce])
                vE = plsc.load_gather(c1b, [row[4], ce])
                w1 = plsc.load_gather(rb, [row[0], ce])
                aA = aA + vA * vA
                aB = aB + vB * vB
                aC = aC + vC * vC
                aD = aD + vD * vD
                aE = aE + vE * vE
                t = vA + w1 - vB
                a1 = a1 + t * t
                t = vD - vC
                aCD = aCD + t * t
                t = vE - vC
                aCE = aCE + t * t
                t = vE - vD
                aDE = aDE + t * t
                return (aA, aB, aC, aD, aE, a1, aCD, aCE, aDE)

            (aA, aB, aC, aD, aE, a1, aCD, aCE, aDE) = plsc.parallel_loop(
                0, EMB, unroll=8, carry=(zero,) * 9)(eb1)

            def eb2(e, accs):
                (aF, aG, aH, aI, aK, aL, a3, a4, a5) = accs
                ce = (jnp.full((L,), e, jnp.int32) + iota) & (EMB - 1)
                vF = plsc.load_gather(c2b, [row[0], ce])
                vG = plsc.load_gather(c2b, [row[1], ce])
                vH = plsc.load_gather(c2b, [row[2], ce])
                vI = plsc.load_gather(c2b, [row[3], ce])
                vK = plsc.load_gather(c2b, [row[4], ce])
                vL = plsc.load_gather(c2b, [row[5], ce])
                w3 = plsc.load_gather(rb, [row[1], ce])
                w4 = plsc.load_gather(rb, [row[2], ce])
                w5 = plsc.load_gather(rb, [row[3], ce])
                aF = aF + vF * vF
                aG = aG + vG * vG
                aH = aH + vH * vH
                aI = aI + vI * vI
                aK = aK + vK * vK
                aL = aL + vL * vL
                t = vF + w3 - vG
                a3 = a3 + t * t
                t = vH - w4 - vI
                a4 = a4 + t * t
                t = vK + w5 - vL
                a5 = a5 + t * t
                return (aF, aG, aH, aI, aK, aL, a3, a4, a5)

            (aF, aG, aH, aI, aK, aL, a3, a4, a5) = plsc.parallel_loop(
                0, EMB, unroll=8, carry=(zero,) * 9)(eb2)

            if bsel == 0:
                @pl.when(g0 == 0)
                def _():
                    drain_t()

            bs = pl.ds(blk * L, L)
            rA = _relu(ta_v[0, bs]); rB = _relu(ta_v[1, bs])
            rC = _relu(ta_v[2, bs]); rD = _relu(ta_v[3, bs])
            rE = _relu(ta_v[4, bs]); rF = _relu(ta_v[5, bs])
            rG = _relu(ta_v[6, bs]); rH = _relu(ta_v[7, bs])
            rI = _relu(ta_v[8, bs]); rK = _relu(ta_v[9, bs])
            rL = _relu(ta_v[10, bs]); rJ = _relu(ta_v[11, bs])
            tP = ta_v[12, bs]

            def reg(a):
                return jnp.abs(_sqrt16(a) - 1.0)

            loss = (
                _relu(_sqrt16(a1) + rA - rB - MARGIN) + reg(aA) + reg(aB)
                + _relu(_sqrt16(aCD) - (rC + rD) - MARGIN)
                + _relu(_sqrt16(aCE) - rC - MARGIN)
                + _relu(_sqrt16(aDE) - rD - MARGIN)
                + _relu(jnp.minimum(rC, rD) - rE - MARGIN)
                + reg(aC) + reg(aD) + reg(aE)
                + _relu(_sqrt16(a3) + rF - rG - MARGIN) + reg(aF) + reg(aG)
                + _relu(_sqrt16(a4) - (rH + rI) - MARGIN) + reg(aH) + reg(aI)
                + jnp.abs(rJ - INF)
                + (MARGIN - (_sqrt16(a5) - rK - rL)) + reg(aK) + reg(aL)
                - jnp.minimum(tP, 0.0)
            )
            tot_v[...] = tot_v[...] + loss

    pltpu.sync_copy(tot_v, out_h.at[wid])


def kernel(cls_emb, rel_emb, nf1, nf2, nf3, nf4, dis, top, nf3_neg,
           nf_inclusion, nf_chain, radius, dataset):
    xs = cls_emb[:, :EMB]
    ts = cls_emb[:, EMB]
    mesh = plsc.VectorSubcoreMesh(core_axis_name="c", subcore_axis_name="s")
    cp = pltpu.CompilerParams(needs_layout_passes=False,
                              use_tc_tiling_on_sc=False)
    sc = pl.kernel(
        _sc_body,
        out_type=jax.ShapeDtypeStruct((NW, L), jnp.float32),
        mesh=mesh,
        compiler_params=cp,
        scratch_types=[
            pltpu.VMEM((BPT, 3), jnp.int32),
            pltpu.VMEM((BPT, 3), jnp.int32),
            pltpu.VMEM((BPT, 3), jnp.int32),
            pltpu.VMEM((BPT, 3), jnp.int32),
            pltpu.VMEM((BPT, 3), jnp.int32),
            pltpu.VMEM((BPT,), jnp.int32),
            pltpu.VMEM((BPT,), jnp.int32),
            pltpu.VMEM((2, NC1 * L), jnp.int32),
            pltpu.VMEM((2, NC2 * L), jnp.int32),
            pltpu.VMEM((2, NR * L), jnp.int32),
            pltpu.VMEM((2, NC1 * L, EMB), jnp.float32),
            pltpu.VMEM((2, NC2 * L, EMB), jnp.float32),
            pltpu.VMEM((2, NR * L, EMB), jnp.float32),
            pltpu.VMEM((11, BPT), jnp.int32),
            pltpu.VMEM((13, BPT), jnp.float32),
            pltpu.VMEM((L,), jnp.float32),
            pltpu.SemaphoreType.DMA,
            pltpu.SemaphoreType.DMA,
        ],
    )
    part = sc(xs, ts, rel_emb,
              nf1.astype(jnp.int32), nf2.astype(jnp.int32),
              nf3.astype(jnp.int32), nf4.astype(jnp.int32),
              top.astype(jnp.int32), nf3_neg.astype(jnp.int32),
              radius.astype(jnp.int32))
    return (jnp.sum(part) / jnp.float32(B)) ** 2


# speedup vs baseline: 1.1436x; 1.1108x over previous
import jax
import jax.numpy as jnp
from jax import lax
from jax.experimental import pallas as pl
from jax.experimental.pallas import tpu as pltpu
from jax.experimental.pallas import tpu_sc as plsc

EMB = 128
MARGIN = 0.1
INF = 5.0
B = 4096
L = 16
NW = 32
BPT = B // NW
NBLK = BPT // L
NC1 = 5
NC2 = 6
NR = 4


def _sqrt16(s):
    i = plsc.bitcast(s, jnp.int32)
    y = plsc.bitcast(jnp.int32(0x5F3759DF) - lax.shift_right_arithmetic(i, 1),
                     jnp.float32)
    for _ in range(3):
        y = y * (1.5 - ((0.5 * s) * y) * y)
    return s * y


def _relu(x):
    return jnp.maximum(x, 0.0)


def _sc_body(xs_hbm, ts_hbm, rel_hbm, nf1_h, nf2_h, nf3_h, nf4_h, top_h,
             nn_h, rad_h, out_h,
             nf1_v, nf2_v, nf3_v, nf4_v, nn_v, top_v, rad_v,
             ic1_v, ic2_v, ir_v, c1_v, c2_v, r_v,
             tc_v, ta_v, tot_v, sem, tsem):
    wid = lax.axis_index("s") * 2 + lax.axis_index("c")
    base = wid * BPT
    iota = lax.iota(jnp.int32, L)

    pltpu.sync_copy(nf1_h.at[pl.ds(base, BPT), :], nf1_v)
    pltpu.sync_copy(nf2_h.at[pl.ds(base, BPT), :], nf2_v)
    pltpu.sync_copy(nf3_h.at[pl.ds(base, BPT), :], nf3_v)
    pltpu.sync_copy(nf4_h.at[pl.ds(base, BPT), :], nf4_v)
    pltpu.sync_copy(nn_h.at[pl.ds(base, BPT), :], nn_v)
    pltpu.sync_copy(top_h.at[pl.ds(base, BPT)], top_v)
    pltpu.sync_copy(rad_h.at[pl.ds(base, BPT)], rad_v)

    col_specs = ((nf1_v, 0), (nf1_v, 2), (nf2_v, 0), (nf2_v, 1), (nf2_v, 2),
                 (nf3_v, 0), (nf3_v, 2), (nf4_v, 1), (nf4_v, 2),
                 (nn_v, 0), (nn_v, 2))
    for s, (ref, c) in enumerate(col_specs):
        cc = jnp.full((L,), c, jnp.int32)
        for b8 in range(NBLK):
            tc_v[s, pl.ds(b8 * L, L)] = plsc.load_gather(
                ref, [iota + b8 * L, cc])

    def build_idx(blk, which):
        rows = iota + blk * L

        def col(ref, c):
            return plsc.load_gather(ref, [rows, jnp.full((L,), c, jnp.int32)])

        ic1_v[which, pl.ds(0 * L, L)] = col(nf1_v, 0)
        ic1_v[which, pl.ds(1 * L, L)] = col(nf1_v, 2)
        ic1_v[which, pl.ds(2 * L, L)] = col(nf2_v, 0)
        ic1_v[which, pl.ds(3 * L, L)] = col(nf2_v, 1)
        ic1_v[which, pl.ds(4 * L, L)] = col(nf2_v, 2)
        ic2_v[which, pl.ds(0 * L, L)] = col(nf3_v, 0)
        ic2_v[which, pl.ds(1 * L, L)] = col(nf3_v, 2)
        ic2_v[which, pl.ds(2 * L, L)] = col(nf4_v, 1)
        ic2_v[which, pl.ds(3 * L, L)] = col(nf4_v, 2)
        ic2_v[which, pl.ds(4 * L, L)] = col(nn_v, 0)
        ic2_v[which, pl.ds(5 * L, L)] = col(nn_v, 2)
        ir_v[which, pl.ds(0 * L, L)] = col(nf1_v, 1)
        ir_v[which, pl.ds(1 * L, L)] = col(nf3_v, 1)
        ir_v[which, pl.ds(2 * L, L)] = col(nf4_v, 0)
        ir_v[which, pl.ds(3 * L, L)] = col(nn_v, 1)

    def issue(which):
        pltpu.async_copy(xs_hbm.at[ic1_v.at[which]], c1_v.at[which], sem)
        pltpu.async_copy(xs_hbm.at[ic2_v.at[which]], c2_v.at[which], sem)
        pltpu.async_copy(rel_hbm.at[ir_v.at[which]], r_v.at[which], sem)

    def drain(which):
        pltpu.make_async_copy(xs_hbm.at[ic1_v.at[which]], c1_v.at[which],
                              sem).wait()
        pltpu.make_async_copy(xs_hbm.at[ic2_v.at[which]], c2_v.at[which],
                              sem).wait()
        pltpu.make_async_copy(rel_hbm.at[ir_v.at[which]], r_v.at[which],
                              sem).wait()

    build_idx(0, 0)
    issue(0)
    tdmas = [pltpu.async_copy(ts_hbm.at[tc_v.at[s]], ta_v.at[s], tsem)
             for s in range(11)]
    tdmas.append(pltpu.async_copy(ts_hbm.at[top_v], ta_v.at[11], tsem))
    tdmas.append(pltpu.async_copy(ts_hbm.at[rad_v], ta_v.at[12], tsem))

    def drain_t():
        for s in range(11):
            pltpu.make_async_copy(ts_hbm.at[tc_v.at[s]], ta_v.at[s],
                                  tsem).wait()
        pltpu.make_async_copy(ts_hbm.at[top_v], ta_v.at[11], tsem).wait()
        pltpu.make_async_copy(ts_hbm.at[rad_v], ta_v.at[12], tsem).wait()

    tot_v[...] = jnp.zeros((L,), jnp.float32)

    @pl.loop(0, NBLK, step=2)
    def _blockpair(g0):
        for bsel in range(2):
            blk = g0 + bsel
            nxt = 1 - bsel
            drain(bsel)
            if bsel == 0:
                build_idx(blk + 1, nxt)
                issue(nxt)
            else:
                @pl.when(g0 + 2 < NBLK)
                def _():
                    build_idx(blk + 1, nxt)
                    issue(nxt)

            zero = jnp.zeros((L,), jnp.float32)
            c1b = c1_v.at[bsel]
            c2b = c2_v.at[bsel]
            rb = r_v.at[bsel]
            row = [iota + s * L for s in range(6)]

            def eb1(e, accs):
                (aA, aB, aC, aD, aE, a1, aCD, aCE, aDE) = accs
                ce = (jnp.full((L,), e, jnp.int32) + iota) & (EMB - 1)
                vA = plsc.load_gather(c1b, [row[0], ce])
                vB = plsc.load_gather(c1b, [row[1], ce])
                vC = plsc.load_gather(c1b, [row[2], ce])
                vD = plsc.load_gather(c1b, [row[3], ce])
                vE = plsc.load_gather(c1b, [row[4], ce])
                w1 = plsc.load_gather(rb, [row[0], ce])
                aA = aA + vA * vA
                aB = aB + vB * vB
                aC = aC + vC * vC
                aD = aD + vD * vD
                aE = aE + vE * vE
                t = vA + w1 - vB
                a1 = a1 + t * t
                t = vD - vC
                aCD = aCD + t * t
                t = vE - vC
                aCE = aCE + t * t
                t = vE - vD
                aDE = aDE + t * t
                return (aA, aB, aC, aD, aE, a1, aCD, aCE, aDE)

            (aA, aB, aC, aD, aE, a1, aCD, aCE, aDE) = plsc.parallel_loop(
                0, EMB, unroll=4, carry=(zero,) * 9)(eb1)

            def eb2(e, accs):
                (aF, aG, aH, aI, aK, aL, a3, a4, a5) = accs
                ce = (jnp.full((L,), e, jnp.int32) + iota) & (EMB - 1)
                vF = plsc.load_gather(c2b, [row[0], ce])
                vG = plsc.load_gather(c2b, [row[1], ce])
                vH = plsc.load_gather(c2b, [row[2], ce])
                vI = plsc.load_gather(c2b, [row[3], ce])
                vK = plsc.load_gather(c2b, [row[4], ce])
                vL = plsc.load_gather(c2b, [row[5], ce])
                w3 = plsc.load_gather(rb, [row[1], ce])
                w4 = plsc.load_gather(rb, [row[2], ce])
                w5 = plsc.load_gather(rb, [row[3], ce])
                aF = aF + vF * vF
                aG = aG + vG * vG
                aH = aH + vH * vH
                aI = aI + vI * vI
                aK = aK + vK * vK
                aL = aL + vL * vL
                t = vF + w3 - vG
                a3 = a3 + t * t
                t = vH - w4 - vI
                a4 = a4 + t * t
                t = vK + w5 - vL
                a5 = a5 + t * t
                return (aF, aG, aH, aI, aK, aL, a3, a4, a5)

            (aF, aG, aH, aI, aK, aL, a3, a4, a5) = plsc.parallel_loop(
                0, EMB, unroll=4, carry=(zero,) * 9)(eb2)

            if bsel == 0:
                @pl.when(g0 == 0)
                def _():
                    drain_t()

            bs = pl.ds(blk * L, L)
            rA = _relu(ta_v[0, bs]); rB = _relu(ta_v[1, bs])
            rC = _relu(ta_v[2, bs]); rD = _relu(ta_v[3, bs])
            rE = _relu(ta_v[4, bs]); rF = _relu(ta_v[5, bs])
            rG = _relu(ta_v[6, bs]); rH = _relu(ta_v[7, bs])
            rI = _relu(ta_v[8, bs]); rK = _relu(ta_v[9, bs])
            rL = _relu(ta_v[10, bs]); rJ = _relu(ta_v[11, bs])
            tP = ta_v[12, bs]

            def reg(a):
                return jnp.abs(_sqrt16(a) - 1.0)

            loss = (
                _relu(_sqrt16(a1) + rA - rB - MARGIN) + reg(aA) + reg(aB)
                + _relu(_sqrt16(aCD) - (rC + rD) - MARGIN)
                + _relu(_sqrt16(aCE) - rC - MARGIN)
                + _relu(_sqrt16(aDE) - rD - MARGIN)
                + _relu(jnp.minimum(rC, rD) - rE - MARGIN)
                + reg(aC) + reg(aD) + reg(aE)
                + _relu(_sqrt16(a3) + rF - rG - MARGIN) + reg(aF) + reg(aG)
                + _relu(_sqrt16(a4) - (rH + rI) - MARGIN) + reg(aH) + reg(aI)
                + jnp.abs(rJ - INF)
                + (MARGIN - (_sqrt16(a5) - rK - rL)) + reg(aK) + reg(aL)
                - jnp.minimum(tP, 0.0)
            )
            tot_v[...] = tot_v[...] + loss

    pltpu.sync_copy(tot_v, out_h.at[wid])


def kernel(cls_emb, rel_emb, nf1, nf2, nf3, nf4, dis, top, nf3_neg,
           nf_inclusion, nf_chain, radius, dataset):
    xs = cls_emb[:, :EMB]
    ts = cls_emb[:, EMB]
    mesh = plsc.VectorSubcoreMesh(core_axis_name="c", subcore_axis_name="s")
    cp = pltpu.CompilerParams(needs_layout_passes=False,
                              use_tc_tiling_on_sc=False)
    sc = pl.kernel(
        _sc_body,
        out_type=jax.ShapeDtypeStruct((NW, L), jnp.float32),
        mesh=mesh,
        compiler_params=cp,
        scratch_types=[
            pltpu.VMEM((BPT, 3), jnp.int32),
            pltpu.VMEM((BPT, 3), jnp.int32),
            pltpu.VMEM((BPT, 3), jnp.int32),
            pltpu.VMEM((BPT, 3), jnp.int32),
            pltpu.VMEM((BPT, 3), jnp.int32),
            pltpu.VMEM((BPT,), jnp.int32),
            pltpu.VMEM((BPT,), jnp.int32),
            pltpu.VMEM((2, NC1 * L), jnp.int32),
            pltpu.VMEM((2, NC2 * L), jnp.int32),
            pltpu.VMEM((2, NR * L), jnp.int32),
            pltpu.VMEM((2, NC1 * L, EMB), jnp.float32),
            pltpu.VMEM((2, NC2 * L, EMB), jnp.float32),
            pltpu.VMEM((2, NR * L, EMB), jnp.float32),
            pltpu.VMEM((11, BPT), jnp.int32),
            pltpu.VMEM((13, BPT), jnp.float32),
            pltpu.VMEM((L,), jnp.float32),
            pltpu.SemaphoreType.DMA,
            pltpu.SemaphoreType.DMA,
        ],
    )
    part = sc(xs, ts, rel_emb,
              nf1.astype(jnp.int32), nf2.astype(jnp.int32),
              nf3.astype(jnp.int32), nf4.astype(jnp.int32),
              top.astype(jnp.int32), nf3_neg.astype(jnp.int32),
              radius.astype(jnp.int32))
    return (jnp.sum(part) / jnp.float32(B)) ** 2


# speedup vs baseline: 1.1656x; 1.0192x over previous
import jax
import jax.numpy as jnp
from jax import lax
from jax.experimental import pallas as pl
from jax.experimental.pallas import tpu as pltpu
from jax.experimental.pallas import tpu_sc as plsc

EMB = 128
MARGIN = 0.1
INF = 5.0
B = 4096
L = 16
NW = 32
BPT = B // NW
NBLK = BPT // L
NC1 = 5
NC2 = 6
NR = 4


def _sqrt16(s):
    i = plsc.bitcast(s, jnp.int32)
    y = plsc.bitcast(jnp.int32(0x5F3759DF) - lax.shift_right_arithmetic(i, 1),
                     jnp.float32)
    for _ in range(3):
        y = y * (1.5 - ((0.5 * s) * y) * y)
    return s * y


def _relu(x):
    return jnp.maximum(x, 0.0)


def _sc_body(xs_hbm, ts_hbm, rel_hbm, nf1_h, nf2_h, nf3_h, nf4_h, top_h,
             nn_h, rad_h, out_h,
             nf1_v, nf2_v, nf3_v, nf4_v, nn_v, top_v, rad_v,
             ic1_v, ic2_v, ir_v, c1_v, c2_v, r_v,
             tc_v, ta_v, tot_v, sem, tsem):
    wid = lax.axis_index("s") * 2 + lax.axis_index("c")
    base = wid * BPT
    iota = lax.iota(jnp.int32, L)

    stage = [
        pltpu.async_copy(nf1_h.at[pl.ds(base, BPT), :], nf1_v, sem),
        pltpu.async_copy(nf2_h.at[pl.ds(base, BPT), :], nf2_v, sem),
        pltpu.async_copy(nf3_h.at[pl.ds(base, BPT), :], nf3_v, sem),
        pltpu.async_copy(nf4_h.at[pl.ds(base, BPT), :], nf4_v, sem),
        pltpu.async_copy(nn_h.at[pl.ds(base, BPT), :], nn_v, sem),
        pltpu.async_copy(top_h.at[pl.ds(base, BPT)], top_v, sem),
        pltpu.async_copy(rad_h.at[pl.ds(base, BPT)], rad_v, sem),
    ]
    for d in stage:
        d.wait()

    col_specs = ((nf1_v, 0), (nf1_v, 2), (nf2_v, 0), (nf2_v, 1), (nf2_v, 2),
                 (nf3_v, 0), (nf3_v, 2), (nf4_v, 1), (nf4_v, 2),
                 (nn_v, 0), (nn_v, 2))
    for s, (ref, c) in enumerate(col_specs):
        cc = jnp.full((L,), c, jnp.int32)
        for b8 in range(NBLK):
            tc_v[s, pl.ds(b8 * L, L)] = plsc.load_gather(
                ref, [iota + b8 * L, cc])

    def build_idx(blk, which):
        rows = iota + blk * L

        def col(ref, c):
            return plsc.load_gather(ref, [rows, jnp.full((L,), c, jnp.int32)])

        ic1_v[which, pl.ds(0 * L, L)] = col(nf1_v, 0)
        ic1_v[which, pl.ds(1 * L, L)] = col(nf1_v, 2)
        ic1_v[which, pl.ds(2 * L, L)] = col(nf2_v, 0)
        ic1_v[which, pl.ds(3 * L, L)] = col(nf2_v, 1)
        ic1_v[which, pl.ds(4 * L, L)] = col(nf2_v, 2)
        ic2_v[which, pl.ds(0 * L, L)] = col(nf3_v, 0)
        ic2_v[which, pl.ds(1 * L, L)] = col(nf3_v, 2)
        ic2_v[which, pl.ds(2 * L, L)] = col(nf4_v, 1)
        ic2_v[which, pl.ds(3 * L, L)] = col(nf4_v, 2)
        ic2_v[which, pl.ds(4 * L, L)] = col(nn_v, 0)
        ic2_v[which, pl.ds(5 * L, L)] = col(nn_v, 2)
        ir_v[which, pl.ds(0 * L, L)] = col(nf1_v, 1)
        ir_v[which, pl.ds(1 * L, L)] = col(nf3_v, 1)
        ir_v[which, pl.ds(2 * L, L)] = col(nf4_v, 0)
        ir_v[which, pl.ds(3 * L, L)] = col(nn_v, 1)

    def issue(which):
        pltpu.async_copy(xs_hbm.at[ic1_v.at[which]], c1_v.at[which], sem)
        pltpu.async_copy(xs_hbm.at[ic2_v.at[which]], c2_v.at[which], sem)
        pltpu.async_copy(rel_hbm.at[ir_v.at[which]], r_v.at[which], sem)

    def drain(which):
        pltpu.make_async_copy(xs_hbm.at[ic1_v.at[which]], c1_v.at[which],
                              sem).wait()
        pltpu.make_async_copy(xs_hbm.at[ic2_v.at[which]], c2_v.at[which],
                              sem).wait()
        pltpu.make_async_copy(rel_hbm.at[ir_v.at[which]], r_v.at[which],
                              sem).wait()

    build_idx(0, 0)
    issue(0)
    tdmas = [pltpu.async_copy(ts_hbm.at[tc_v.at[s]], ta_v.at[s], tsem)
             for s in range(11)]
    tdmas.append(pltpu.async_copy(ts_hbm.at[top_v], ta_v.at[11], tsem))
    tdmas.append(pltpu.async_copy(ts_hbm.at[rad_v], ta_v.at[12], tsem))

    def drain_t():
        for s in range(11):
            pltpu.make_async_copy(ts_hbm.at[tc_v.at[s]], ta_v.at[s],
                                  tsem).wait()
        pltpu.make_async_copy(ts_hbm.at[top_v], ta_v.at[11], tsem).wait()
        pltpu.make_async_copy(ts_hbm.at[rad_v], ta_v.at[12], tsem).wait()

    tot_v[...] = jnp.zeros((L,), jnp.float32)

    @pl.loop(0, NBLK, step=2)
    def _blockpair(g0):
        for bsel in range(2):
            blk = g0 + bsel
            nxt = 1 - bsel
            drain(bsel)
            if bsel == 0:
                build_idx(blk + 1, nxt)
                issue(nxt)
            else:
                @pl.when(g0 + 2 < NBLK)
                def _():
                    build_idx(blk + 1, nxt)
                    issue(nxt)

            zero = jnp.zeros((L,), jnp.float32)
            c1b = c1_v.at[bsel]
            c2b = c2_v.at[bsel]
            rb = r_v.at[bsel]
            row = [iota + s * L for s in range(6)]

            def eb1(e, accs):
                (aA, aB, aC, aD, aE, a1, aCD, aCE, aDE) = accs
                ce = (jnp.full((L,), e, jnp.int32) + iota) & (EMB - 1)
                vA = plsc.load_gather(c1b, [row[0], ce])
                vB = plsc.load_gather(c1b, [row[1], ce])
                vC = plsc.load_gather(c1b, [row[2], ce])
                vD = plsc.load_gather(c1b, [row[3], ce])
                vE = plsc.load_gather(c1b, [row[4], ce])
                w1 = plsc.load_gather(rb, [row[0], ce])
                aA = aA + vA * vA
                aB = aB + vB * vB
                aC = aC + vC * vC
                aD = aD + vD * vD
                aE = aE + vE * vE
                t = vA + w1 - vB
                a1 = a1 + t * t
                t = vD - vC
                aCD = aCD + t * t
                t = vE - vC
                aCE = aCE + t * t
                t = vE - vD
                aDE = aDE + t * t
                return (aA, aB, aC, aD, aE, a1, aCD, aCE, aDE)

            (aA, aB, aC, aD, aE, a1, aCD, aCE, aDE) = plsc.parallel_loop(
                0, EMB, unroll=4, carry=(zero,) * 9)(eb1)

            def eb2(e, accs):
                (aF, aG, aH, aI, aK, aL, a3, a4, a5) = accs
                ce = (jnp.full((L,), e, jnp.int32) + iota) & (EMB - 1)
                vF = plsc.load_gather(c2b, [row[0], ce])
                vG = plsc.load_gather(c2b, [row[1], ce])
                vH = plsc.load_gather(c2b, [row[2], ce])
                vI = plsc.load_gather(c2b, [row[3], ce])
                vK = plsc.load_gather(c2b, [row[4], ce])
                vL = plsc.load_gather(c2b, [row[5], ce])
                w3 = plsc.load_gather(rb, [row[1], ce])
                w4 = plsc.load_gather(rb, [row[2], ce])
                w5 = plsc.load_gather(rb, [row[3], ce])
                aF = aF + vF * vF
                aG = aG + vG * vG
                aH = aH + vH * vH
                aI = aI + vI * vI
                aK = aK + vK * vK
                aL = aL + vL * vL
                t = vF + w3 - vG
                a3 = a3 + t * t
                t = vH - w4 - vI
                a4 = a4 + t * t
                t = vK + w5 - vL
                a5 = a5 + t * t
                return (aF, aG, aH, aI, aK, aL, a3, a4, a5)

            (aF, aG, aH, aI, aK, aL, a3, a4, a5) = plsc.parallel_loop(
                0, EMB, unroll=4, carry=(zero,) * 9)(eb2)

            if bsel == 0:
                @pl.when(g0 == 0)
                def _():
                    drain_t()

            bs = pl.ds(blk * L, L)
            rA = _relu(ta_v[0, bs]); rB = _relu(ta_v[1, bs])
            rC = _relu(ta_v[2, bs]); rD = _relu(ta_v[3, bs])
            rE = _relu(ta_v[4, bs]); rF = _relu(ta_v[5, bs])
            rG = _relu(ta_v[6, bs]); rH = _relu(ta_v[7, bs])
            rI = _relu(ta_v[8, bs]); rK = _relu(ta_v[9, bs])
            rL = _relu(ta_v[10, bs]); rJ = _relu(ta_v[11, bs])
            tP = ta_v[12, bs]

            def reg(a):
                return jnp.abs(_sqrt16(a) - 1.0)

            loss = (
                _relu(_sqrt16(a1) + rA - rB - MARGIN) + reg(aA) + reg(aB)
                + _relu(_sqrt16(aCD) - (rC + rD) - MARGIN)
                + _relu(_sqrt16(aCE) - rC - MARGIN)
                + _relu(_sqrt16(aDE) - rD - MARGIN)
                + _relu(jnp.minimum(rC, rD) - rE - MARGIN)
                + reg(aC) + reg(aD) + reg(aE)
                + _relu(_sqrt16(a3) + rF - rG - MARGIN) + reg(aF) + reg(aG)
                + _relu(_sqrt16(a4) - (rH + rI) - MARGIN) + reg(aH) + reg(aI)
                + jnp.abs(rJ - INF)
                + (MARGIN - (_sqrt16(a5) - rK - rL)) + reg(aK) + reg(aL)
                - jnp.minimum(tP, 0.0)
            )
            tot_v[...] = tot_v[...] + loss

    pltpu.sync_copy(tot_v, out_h.at[wid])


def kernel(cls_emb, rel_emb, nf1, nf2, nf3, nf4, dis, top, nf3_neg,
           nf_inclusion, nf_chain, radius, dataset):
    xs = cls_emb[:, :EMB]
    ts = cls_emb[:, EMB]
    mesh = plsc.VectorSubcoreMesh(core_axis_name="c", subcore_axis_name="s")
    cp = pltpu.CompilerParams(needs_layout_passes=False,
                              use_tc_tiling_on_sc=False)
    sc = pl.kernel(
        _sc_body,
        out_type=jax.ShapeDtypeStruct((NW, L), jnp.float32),
        mesh=mesh,
        compiler_params=cp,
        scratch_types=[
            pltpu.VMEM((BPT, 3), jnp.int32),
            pltpu.VMEM((BPT, 3), jnp.int32),
            pltpu.VMEM((BPT, 3), jnp.int32),
            pltpu.VMEM((BPT, 3), jnp.int32),
            pltpu.VMEM((BPT, 3), jnp.int32),
            pltpu.VMEM((BPT,), jnp.int32),
            pltpu.VMEM((BPT,), jnp.int32),
            pltpu.VMEM((2, NC1 * L), jnp.int32),
            pltpu.VMEM((2, NC2 * L), jnp.int32),
            pltpu.VMEM((2, NR * L), jnp.int32),
            pltpu.VMEM((2, NC1 * L, EMB), jnp.float32),
            pltpu.VMEM((2, NC2 * L, EMB), jnp.float32),
            pltpu.VMEM((2, NR * L, EMB), jnp.float32),
            pltpu.VMEM((11, BPT), jnp.int32),
            pltpu.VMEM((13, BPT), jnp.float32),
            pltpu.VMEM((L,), jnp.float32),
            pltpu.SemaphoreType.DMA,
            pltpu.SemaphoreType.DMA,
        ],
    )
    part = sc(xs, ts, rel_emb,
              nf1.astype(jnp.int32), nf2.astype(jnp.int32),
              nf3.astype(jnp.int32), nf4.astype(jnp.int32),
              top.astype(jnp.int32), nf3_neg.astype(jnp.int32),
              radius.astype(jnp.int32))
    return (jnp.sum(part) / jnp.float32(B)) ** 2
